# baseline (speedup 1.0000x reference)
import jax
import jax.numpy as jnp
from jax import lax
from jax.experimental import pallas as pl
from jax.experimental.pallas import tpu as pltpu
from jax.experimental.pallas import tpu_sc as plsc

_BLK = 2048
_GROWS = 16


def _sc_lookup_body(w_hbm, out_hbm, idx_v, rows_v, sem):
    cid = lax.axis_index("c")
    sid = lax.axis_index("s")

    @pl.when(jnp.logical_and(cid == 0, sid == 0))
    def _():
        idx_v[...] = lax.iota(jnp.int32, _GROWS)
        pltpu.async_copy(w_hbm.at[idx_v], rows_v, sem).wait()
        pltpu.sync_copy(rows_v, out_hbm)


def _sc_lookup(W):
    D = W.shape[-1]
    mesh = plsc.VectorSubcoreMesh(core_axis_name="c", subcore_axis_name="s")
    return pl.kernel(
        _sc_lookup_body,
        mesh=mesh,
        out_type=jax.ShapeDtypeStruct((_GROWS, D), jnp.float32),
        scratch_types=[
            pltpu.VMEM((_GROWS,), jnp.int32),
            pltpu.VMEM((_GROWS, D), jnp.float32),
            pltpu.SemaphoreType.DMA,
        ],
    )(W)


def _ln_kernel(x_ref, w_ref, g_ref, b_ref, o_ref):
    x = x_ref[0]
    e = w_ref[0, 0]
    y = x + e[None, :]
    mean = jnp.mean(y, axis=1, keepdims=True)
    yc = y - mean
    var = jnp.mean(yc * yc, axis=1, keepdims=True)
    inv = jax.lax.rsqrt(var + 1e-9)
    o_ref[0] = yc * inv * g_ref[0][None, :] + b_ref[0][None, :]


def kernel(x, W, gamma, beta):
    B, S, D = x.shape
    emb = _sc_lookup(W).reshape(_GROWS, 1, D)
    g2 = gamma.reshape(1, D)
    b2 = beta.reshape(1, D)
    grid = (B, S // _BLK)
    return pl.pallas_call(
        _ln_kernel,
        grid=grid,
        in_specs=[
            pl.BlockSpec((1, _BLK, D), lambda b, s: (b, s, 0)),
            pl.BlockSpec((1, 1, D), lambda b, s: (b, 0, 0)),
            pl.BlockSpec((1, D), lambda b, s: (0, 0)),
            pl.BlockSpec((1, D), lambda b, s: (0, 0)),
        ],
        out_specs=pl.BlockSpec((1, _BLK, D), lambda b, s: (b, s, 0)),
        out_shape=jax.ShapeDtypeStruct((B, S, D), x.dtype),
        compiler_params=pltpu.CompilerParams(
            dimension_semantics=("parallel", "parallel"),
        ),
    )(x, emb, g2, b2)
